# baseline (speedup 1.0000x reference)
import functools

import jax
import jax.numpy as jnp
from jax import lax
from jax.experimental import pallas as pl
from jax.experimental.pallas import tpu as pltpu
from jax.experimental.pallas import tpu_sc as plsc

_NEG = -3.0e38
_BIG = 1 << 30


def _gate_body(h_ref, wg_ref, out_ref):
    h = h_ref[0]
    w = wg_ref[...]
    out_ref[0] = lax.dot_general(
        h, w, (((1,), (1,)), ((), ())), preferred_element_type=jnp.float32
    )


def _gate_call(hidden, w_gate):
    B, S, H = hidden.shape
    NH = w_gate.shape[0]
    SB = 512
    return pl.pallas_call(
        _gate_body,
        grid=(B, S // SB),
        in_specs=[
            pl.BlockSpec((1, SB, H), lambda b, s: (b, s, 0)),
            pl.BlockSpec((NH, H), lambda b, s: (0, 0)),
        ],
        out_specs=pl.BlockSpec((1, SB, NH), lambda b, s: (b, s, 0)),
        out_shape=jax.ShapeDtypeStruct((B, S, NH), jnp.float32),
    )(hidden, w_gate)


def _topk_gather_mask_call(gate_rows, hidden2, k, B, NH):
    R, S = gate_rows.shape
    _, H = hidden2.shape
    NGRP = 16
    GSZ = S // NGRP
    NV = GSZ // 16
    CH = 16
    NBUF = 2
    BPC = B // 2

    mesh = plsc.VectorSubcoreMesh(core_axis_name="c", subcore_axis_name="s")

    @functools.partial(
        pl.kernel,
        mesh=mesh,
        compiler_params=pltpu.CompilerParams(needs_layout_passes=False),
        out_type=(
            jax.ShapeDtypeStruct((NH, B * k, H), jnp.float32),
            jax.ShapeDtypeStruct((R, k), jnp.int32),
        ),
        scratch_types=[
            pltpu.VMEM((S,), jnp.float32),
            pltpu.VMEM((NGRP * 16,), jnp.float32),
            pltpu.VMEM((NGRP * 16,), jnp.int32),
            pltpu.VMEM((k,), jnp.int32),
            pltpu.VMEM((k,), jnp.int32),
            pltpu.VMEM((NBUF, CH, H), jnp.float32),
            pltpu.SemaphoreType.DMA,
            pltpu.SemaphoreType.DMA,
        ],
    )
    def topk_kernel(gate_hbm, hid_hbm, g2_hbm, idxo_hbm,
                    row_v, vg_v, ig_v, idx_v, gidx_v, gbuf_v, sem0, sem1):
        c = lax.axis_index("c")
        s = lax.axis_index("s")
        b = c * BPC + s // NH
        h = s - (s // NH) * NH
        wid = b * NH + h
        pltpu.sync_copy(gate_hbm.at[wid], row_v)
        iota = lax.broadcasted_iota(jnp.int32, (16,), 0)

        for g in range(NGRP):
            base = g * GSZ
            V = row_v[pl.ds(base, 16)]
            I = iota + base
            for j in range(1, NV):
                x = row_v[pl.ds(base + j * 16, 16)]
                ii = iota + (base + j * 16)
                gt = x > V
                V = jnp.where(gt, x, V)
                I = jnp.where(gt, ii, I)
            vg_v[pl.ds(g * 16, 16)] = V
            ig_v[pl.ds(g * 16, 16)] = I

        def select_step(t, _):
            M = vg_v[pl.ds(0, 16)]
            J = ig_v[pl.ds(0, 16)]
            for g in range(1, NGRP):
                V = vg_v[pl.ds(g * 16, 16)]
                I = ig_v[pl.ds(g * 16, 16)]
                gt = V > M
                M = jnp.where(gt, V, M)
                J = jnp.where(gt, I, J)
            m = jnp.max(M)
            idx = jnp.min(jnp.where(M == m, J, _BIG))
            pb = (t // 16) * 16
            ov = idx_v[pl.ds(pb, 16)]
            idx_v[pl.ds(pb, 16)] = jnp.where(iota == (t - pb), idx, ov)
            vb = (idx // 16) * 16
            x = row_v[pl.ds(vb, 16)]
            row_v[pl.ds(vb, 16)] = jnp.where((iota + vb) == idx, _NEG, x)
            gsel = idx // GSZ
            gb = gsel * GSZ
            V2 = row_v[pl.ds(gb, 16)]
            I2 = iota + gb
            for j in range(1, NV):
                x2 = row_v[pl.ds(gb + j * 16, 16)]
                ii2 = iota + gb + j * 16
                gt2 = x2 > V2
                V2 = jnp.where(gt2, x2, V2)
                I2 = jnp.where(gt2, ii2, I2)
            vg_v[pl.ds(gsel * 16, 16)] = V2
            ig_v[pl.ds(gsel * 16, 16)] = I2
            return 0

        lax.fori_loop(0, k, select_step, 0)

        pltpu.sync_copy(idx_v, idxo_hbm.at[wid])

        for j in range(k // 16):
            gidx_v[pl.ds(j * 16, 16)] = idx_v[pl.ds(j * 16, 16)] + b * S

        nch = k // CH
        sems = [sem0, sem1]

        def start(ci):
            return pltpu.async_copy(
                hid_hbm.at[gidx_v.at[pl.ds(ci * CH, CH)]],
                gbuf_v.at[ci % NBUF],
                sems[ci % NBUF],
            )

        copies = [start(ci) for ci in range(min(NBUF, nch))]
        for ci in range(nch):
            copies[ci].wait()
            pltpu.sync_copy(
                gbuf_v.at[ci % NBUF],
                g2_hbm.at[h, pl.ds(b * k + ci * CH, CH), :],
            )
            if ci + NBUF < nch:
                copies.append(start(ci + NBUF))

    return topk_kernel(gate_rows, hidden2)


def _mask_call(idx_bf, S):
    B, NK = idx_bf.shape

    mesh = plsc.VectorSubcoreMesh(core_axis_name="c", subcore_axis_name="s")

    @functools.partial(
        pl.kernel,
        mesh=mesh,
        compiler_params=pltpu.CompilerParams(needs_layout_passes=False),
        out_type=jax.ShapeDtypeStruct((B, S), jnp.float32),
        scratch_types=[
            pltpu.VMEM((S,), jnp.float32),
            pltpu.VMEM((NK,), jnp.int32),
        ],
    )
    def mask_kernel(idx_hbm, out_hbm, mask_v, sidx_v):
        wid = lax.axis_index("s") * 2 + lax.axis_index("c")

        @pl.when(wid < B)
        def _():
            zeros = jnp.zeros((16,), jnp.float32)
            ones = jnp.ones((16,), jnp.float32)

            def zstep(i, _):
                mask_v[pl.ds(i * 16, 16)] = zeros
                return 0

            lax.fori_loop(0, S // 16, zstep, 0)
            pltpu.sync_copy(idx_hbm.at[wid], sidx_v)

            def sstep(i, _):
                iv = sidx_v[pl.ds(i * 16, 16)]
                plsc.store_scatter(mask_v, [iv], ones)
                return 0

            lax.fori_loop(0, NK // 16, sstep, 0)
            pltpu.sync_copy(mask_v, out_hbm.at[wid])

    return mask_kernel(idx_bf)


def _proj_body(g_ref, wp_ref, out_ref):
    g = g_ref[0]
    w = wp_ref[0]
    o = lax.dot_general(
        g, w, (((1,), (1,)), ((), ())), preferred_element_type=jnp.float32
    )
    out_ref[...] = o.reshape(out_ref.shape)


def _proj_call(g2, wp, B, k):
    NH, Bk, H = g2.shape
    HD = wp.shape[1]
    return pl.pallas_call(
        _proj_body,
        grid=(NH,),
        in_specs=[
            pl.BlockSpec((1, Bk, H), lambda h: (h, 0, 0)),
            pl.BlockSpec((1, HD, H), lambda h: (h, 0, 0)),
        ],
        out_specs=pl.BlockSpec((B, 1, k, HD), lambda h: (0, h, 0, 0)),
        out_shape=jax.ShapeDtypeStruct((B, NH, k, HD), jnp.float32),
    )(g2, wp)


def kernel(hidden_states, W_proj, W_gate):
    B, S, H = hidden_states.shape
    NH = W_gate.shape[0]
    HD = W_proj.shape[0] // NH
    K = min(64, S)

    gate_bsn = _gate_call(hidden_states, W_gate)
    gate_rows = jnp.transpose(gate_bsn, (0, 2, 1)).reshape(B * NH, S)

    g2, topk_idx = _topk_gather_mask_call(
        gate_rows, hidden_states.reshape(B * S, H), K, B, NH
    )

    mask_f = _mask_call(topk_idx.reshape(B, NH * K), S)
    index_mask = mask_f != 0.0

    out4 = _proj_call(g2, W_proj.reshape(NH, HD, H), B, K)
    return out4.reshape(B, NH * K, HD), index_mask


# speedup vs baseline: 1.1168x; 1.1168x over previous
import functools

import jax
import jax.numpy as jnp
from jax import lax
from jax.experimental import pallas as pl
from jax.experimental.pallas import tpu as pltpu
from jax.experimental.pallas import tpu_sc as plsc

_NEG = -3.0e38
_BIG = 1 << 30


def _gate_body(h_ref, wg_ref, out_ref):
    h = h_ref[0]
    w = wg_ref[...]
    out_ref[0] = lax.dot_general(
        h, w, (((1,), (1,)), ((), ())), preferred_element_type=jnp.float32
    )


def _gate_call(hidden, w_gate):
    B, S, H = hidden.shape
    NH = w_gate.shape[0]
    SB = 1024
    return pl.pallas_call(
        _gate_body,
        grid=(B, S // SB),
        in_specs=[
            pl.BlockSpec((1, SB, H), lambda b, s: (b, s, 0)),
            pl.BlockSpec((NH, H), lambda b, s: (0, 0)),
        ],
        out_specs=pl.BlockSpec((1, SB, NH), lambda b, s: (b, s, 0)),
        out_shape=jax.ShapeDtypeStruct((B, S, NH), jnp.float32),
    )(hidden, w_gate)


def _topk_gather_mask_call(gate_rows, hidden2, k, B, NH):
    R, S = gate_rows.shape
    _, H = hidden2.shape
    NGRP = 16
    GSZ = S // NGRP
    NV = GSZ // 16
    CH = 16
    NBUF = 2
    BPC = B // 2

    mesh = plsc.VectorSubcoreMesh(core_axis_name="c", subcore_axis_name="s")

    @functools.partial(
        pl.kernel,
        mesh=mesh,
        compiler_params=pltpu.CompilerParams(needs_layout_passes=False),
        out_type=(
            jax.ShapeDtypeStruct((NH, B * k, H), jnp.float32),
            jax.ShapeDtypeStruct((R, k), jnp.int32),
        ),
        scratch_types=[
            pltpu.VMEM((S,), jnp.float32),
            pltpu.VMEM((NGRP * 16,), jnp.float32),
            pltpu.VMEM((NGRP * 16,), jnp.int32),
            pltpu.VMEM((k,), jnp.int32),
            pltpu.VMEM((k,), jnp.int32),
            pltpu.VMEM((NBUF, CH, H), jnp.float32),
            pltpu.SemaphoreType.DMA,
            pltpu.SemaphoreType.DMA,
        ],
    )
    def topk_kernel(gate_hbm, hid_hbm, g2_hbm, idxo_hbm,
                    row_v, vg_v, ig_v, idx_v, gidx_v, gbuf_v, sem0, sem1):
        c = lax.axis_index("c")
        s = lax.axis_index("s")
        b = c * BPC + s // NH
        h = s - (s // NH) * NH
        wid = b * NH + h
        pltpu.sync_copy(gate_hbm.at[wid], row_v)
        iota = lax.broadcasted_iota(jnp.int32, (16,), 0)

        for g in range(NGRP):
            base = g * GSZ
            V = row_v[pl.ds(base, 16)]
            I = iota + base
            for j in range(1, NV):
                x = row_v[pl.ds(base + j * 16, 16)]
                ii = iota + (base + j * 16)
                gt = x > V
                V = jnp.where(gt, x, V)
                I = jnp.where(gt, ii, I)
            vg_v[pl.ds(g * 16, 16)] = V
            ig_v[pl.ds(g * 16, 16)] = I

        def select_step(t, _):
            M = vg_v[pl.ds(0, 16)]
            J = ig_v[pl.ds(0, 16)]
            for g in range(1, NGRP):
                V = vg_v[pl.ds(g * 16, 16)]
                I = ig_v[pl.ds(g * 16, 16)]
                gt = V > M
                M = jnp.where(gt, V, M)
                J = jnp.where(gt, I, J)
            m = jnp.max(M)
            idx = jnp.min(jnp.where(M == m, J, _BIG))
            pb = (t // 16) * 16
            ov = idx_v[pl.ds(pb, 16)]
            idx_v[pl.ds(pb, 16)] = jnp.where(iota == (t - pb), idx, ov)
            vb = (idx // 16) * 16
            x = row_v[pl.ds(vb, 16)]
            row_v[pl.ds(vb, 16)] = jnp.where((iota + vb) == idx, _NEG, x)
            gsel = idx // GSZ
            gb = gsel * GSZ
            V2 = row_v[pl.ds(gb, 16)]
            I2 = iota + gb
            for j in range(1, NV):
                x2 = row_v[pl.ds(gb + j * 16, 16)]
                ii2 = iota + gb + j * 16
                gt2 = x2 > V2
                V2 = jnp.where(gt2, x2, V2)
                I2 = jnp.where(gt2, ii2, I2)
            vg_v[pl.ds(gsel * 16, 16)] = V2
            ig_v[pl.ds(gsel * 16, 16)] = I2
            return 0

        nch = k // CH
        sems = [sem0, sem1]
        copies = []

        def start(ci):
            return pltpu.async_copy(
                hid_hbm.at[gidx_v.at[pl.ds(ci * CH, CH)]],
                gbuf_v.at[ci % NBUF],
                sems[ci % NBUF],
            )

        for ci in range(nch):
            lax.fori_loop(ci * CH, (ci + 1) * CH, select_step, 0)
            gidx_v[pl.ds(ci * CH, 16)] = idx_v[pl.ds(ci * CH, 16)] + b * S
            if ci >= NBUF:
                copies[ci - NBUF].wait()
                pltpu.sync_copy(
                    gbuf_v.at[(ci - NBUF) % NBUF],
                    g2_hbm.at[h, pl.ds(b * k + (ci - NBUF) * CH, CH), :],
                )
            copies.append(start(ci))
        for ci in range(nch - NBUF, nch):
            copies[ci].wait()
            pltpu.sync_copy(
                gbuf_v.at[ci % NBUF],
                g2_hbm.at[h, pl.ds(b * k + ci * CH, CH), :],
            )
        pltpu.sync_copy(idx_v, idxo_hbm.at[wid])

    return topk_kernel(gate_rows, hidden2)


def _mask_call(idx_bf, S):
    B, NK = idx_bf.shape

    mesh = plsc.VectorSubcoreMesh(core_axis_name="c", subcore_axis_name="s")

    @functools.partial(
        pl.kernel,
        mesh=mesh,
        compiler_params=pltpu.CompilerParams(needs_layout_passes=False),
        out_type=jax.ShapeDtypeStruct((B, S), jnp.float32),
        scratch_types=[
            pltpu.VMEM((S,), jnp.float32),
            pltpu.VMEM((NK,), jnp.int32),
        ],
    )
    def mask_kernel(idx_hbm, out_hbm, mask_v, sidx_v):
        wid = lax.axis_index("s") * 2 + lax.axis_index("c")

        @pl.when(wid < B)
        def _():
            zeros = jnp.zeros((16,), jnp.float32)
            ones = jnp.ones((16,), jnp.float32)

            def zstep(i, _):
                mask_v[pl.ds(i * 16, 16)] = zeros
                return 0

            lax.fori_loop(0, S // 16, zstep, 0)
            pltpu.sync_copy(idx_hbm.at[wid], sidx_v)

            def sstep(i, _):
                iv = sidx_v[pl.ds(i * 16, 16)]
                plsc.store_scatter(mask_v, [iv], ones)
                return 0

            lax.fori_loop(0, NK // 16, sstep, 0)
            pltpu.sync_copy(mask_v, out_hbm.at[wid])

    return mask_kernel(idx_bf)


def _proj_body(g_ref, wp_ref, out_ref):
    g = g_ref[0]
    w = wp_ref[0]
    o = lax.dot_general(
        g, w, (((1,), (1,)), ((), ())), preferred_element_type=jnp.float32
    )
    out_ref[...] = o.reshape(out_ref.shape)


def _proj_call(g2, wp, B, k):
    NH, Bk, H = g2.shape
    HD = wp.shape[1]
    return pl.pallas_call(
        _proj_body,
        grid=(NH,),
        in_specs=[
            pl.BlockSpec((1, Bk, H), lambda h: (h, 0, 0)),
            pl.BlockSpec((1, HD, H), lambda h: (h, 0, 0)),
        ],
        out_specs=pl.BlockSpec((B, 1, k, HD), lambda h: (0, h, 0, 0)),
        out_shape=jax.ShapeDtypeStruct((B, NH, k, HD), jnp.float32),
    )(g2, wp)


def kernel(hidden_states, W_proj, W_gate):
    B, S, H = hidden_states.shape
    NH = W_gate.shape[0]
    HD = W_proj.shape[0] // NH
    K = min(64, S)

    gate_bsn = _gate_call(hidden_states, W_gate)
    gate_rows = jnp.transpose(gate_bsn, (0, 2, 1)).reshape(B * NH, S)

    g2, topk_idx = _topk_gather_mask_call(
        gate_rows, hidden_states.reshape(B * S, H), K, B, NH
    )

    mask_f = _mask_call(topk_idx.reshape(B, NH * K), S)
    index_mask = mask_f != 0.0

    out4 = _proj_call(g2, W_proj.reshape(NH, HD, H), B, K)
    return out4.reshape(B, NH * K, HD), index_mask


# speedup vs baseline: 1.2139x; 1.0869x over previous
import functools

import jax
import jax.numpy as jnp
from jax import lax
from jax.experimental import pallas as pl
from jax.experimental.pallas import tpu as pltpu
from jax.experimental.pallas import tpu_sc as plsc

_NEG = -3.0e38
_BIG = 1 << 30


def _gate_body(h_ref, wg_ref, out_ref):
    h = h_ref[0]
    w = wg_ref[...]
    o = lax.dot_general(
        h, w, (((1,), (1,)), ((), ())), preferred_element_type=jnp.float32
    )
    out_ref[0] = o.T


def _gate_call(hidden, w_gate):
    B, S, H = hidden.shape
    NH = w_gate.shape[0]
    SB = 1024
    return pl.pallas_call(
        _gate_body,
        grid=(B, S // SB),
        in_specs=[
            pl.BlockSpec((1, SB, H), lambda b, s: (b, s, 0)),
            pl.BlockSpec((NH, H), lambda b, s: (0, 0)),
        ],
        out_specs=pl.BlockSpec((1, NH, SB), lambda b, s: (b, 0, s)),
        out_shape=jax.ShapeDtypeStruct((B, NH, S), jnp.float32),
    )(hidden, w_gate)


def _topk_gather_mask_call(gate_rows, hidden2, k, B, NH):
    R, S = gate_rows.shape
    _, H = hidden2.shape
    NGRP = 16
    GSZ = S // NGRP
    NV = GSZ // 16
    CH = 16
    NBUF = 2
    BPC = B // 2

    mesh = plsc.VectorSubcoreMesh(core_axis_name="c", subcore_axis_name="s")

    @functools.partial(
        pl.kernel,
        mesh=mesh,
        compiler_params=pltpu.CompilerParams(needs_layout_passes=False),
        out_type=(
            jax.ShapeDtypeStruct((NH, B * k, H), jnp.float32),
            jax.ShapeDtypeStruct((R, k), jnp.int32),
        ),
        scratch_types=[
            pltpu.VMEM((S,), jnp.float32),
            pltpu.VMEM((NGRP * 16,), jnp.float32),
            pltpu.VMEM((NGRP * 16,), jnp.int32),
            pltpu.VMEM((k,), jnp.int32),
            pltpu.VMEM((k,), jnp.int32),
            pltpu.VMEM((NBUF, CH, H), jnp.float32),
            pltpu.SemaphoreType.DMA,
            pltpu.SemaphoreType.DMA,
        ],
    )
    def topk_kernel(gate_hbm, hid_hbm, g2_hbm, idxo_hbm,
                    row_v, vg_v, ig_v, idx_v, gidx_v, gbuf_v, sem0, sem1):
        c = lax.axis_index("c")
        s = lax.axis_index("s")
        b = c * BPC + s // NH
        h = s - (s // NH) * NH
        wid = b * NH + h
        pltpu.sync_copy(gate_hbm.at[wid], row_v)
        iota = lax.broadcasted_iota(jnp.int32, (16,), 0)

        for g in range(NGRP):
            base = g * GSZ
            V = row_v[pl.ds(base, 16)]
            I = iota + base
            for j in range(1, NV):
                x = row_v[pl.ds(base + j * 16, 16)]
                ii = iota + (base + j * 16)
                gt = x > V
                V = jnp.where(gt, x, V)
                I = jnp.where(gt, ii, I)
            vg_v[pl.ds(g * 16, 16)] = V
            ig_v[pl.ds(g * 16, 16)] = I

        def select_step(t, _):
            M = vg_v[pl.ds(0, 16)]
            J = ig_v[pl.ds(0, 16)]
            for g in range(1, NGRP):
                V = vg_v[pl.ds(g * 16, 16)]
                I = ig_v[pl.ds(g * 16, 16)]
                gt = V > M
                M = jnp.where(gt, V, M)
                J = jnp.where(gt, I, J)
            m = jnp.max(M)
            idx = jnp.min(jnp.where(M == m, J, _BIG))
            pb = (t // 16) * 16
            ov = idx_v[pl.ds(pb, 16)]
            idx_v[pl.ds(pb, 16)] = jnp.where(iota == (t - pb), idx, ov)
            vb = (idx // 16) * 16
            x = row_v[pl.ds(vb, 16)]
            row_v[pl.ds(vb, 16)] = jnp.where((iota + vb) == idx, _NEG, x)
            gsel = idx // GSZ
            gb = gsel * GSZ
            V2 = row_v[pl.ds(gb, 16)]
            I2 = iota + gb
            for j in range(1, NV):
                x2 = row_v[pl.ds(gb + j * 16, 16)]
                ii2 = iota + gb + j * 16
                gt2 = x2 > V2
                V2 = jnp.where(gt2, x2, V2)
                I2 = jnp.where(gt2, ii2, I2)
            vg_v[pl.ds(gsel * 16, 16)] = V2
            ig_v[pl.ds(gsel * 16, 16)] = I2
            return 0

        nch = k // CH
        sems = [sem0, sem1]
        copies = []

        def start(ci):
            return pltpu.async_copy(
                hid_hbm.at[gidx_v.at[pl.ds(ci * CH, CH)]],
                gbuf_v.at[ci % NBUF],
                sems[ci % NBUF],
            )

        for ci in range(nch):
            lax.fori_loop(ci * CH, (ci + 1) * CH, select_step, 0)
            gidx_v[pl.ds(ci * CH, 16)] = idx_v[pl.ds(ci * CH, 16)] + b * S
            if ci >= NBUF:
                copies[ci - NBUF].wait()
                pltpu.sync_copy(
                    gbuf_v.at[(ci - NBUF) % NBUF],
                    g2_hbm.at[h, pl.ds(b * k + (ci - NBUF) * CH, CH), :],
                )
            copies.append(start(ci))
        for ci in range(nch - NBUF, nch):
            copies[ci].wait()
            pltpu.sync_copy(
                gbuf_v.at[ci % NBUF],
                g2_hbm.at[h, pl.ds(b * k + ci * CH, CH), :],
            )
        pltpu.sync_copy(idx_v, idxo_hbm.at[wid])

    return topk_kernel(gate_rows, hidden2)


def _mask_call(idx_bf, S):
    B, NK = idx_bf.shape

    mesh = plsc.VectorSubcoreMesh(core_axis_name="c", subcore_axis_name="s")

    @functools.partial(
        pl.kernel,
        mesh=mesh,
        compiler_params=pltpu.CompilerParams(needs_layout_passes=False),
        out_type=jax.ShapeDtypeStruct((B, S), jnp.float32),
        scratch_types=[
            pltpu.VMEM((S,), jnp.float32),
            pltpu.VMEM((NK,), jnp.int32),
        ],
    )
    def mask_kernel(idx_hbm, out_hbm, mask_v, sidx_v):
        wid = lax.axis_index("s") * 2 + lax.axis_index("c")

        @pl.when(wid < B)
        def _():
            zeros = jnp.zeros((16,), jnp.float32)
            ones = jnp.ones((16,), jnp.float32)

            def zstep(i, _):
                mask_v[pl.ds(i * 16, 16)] = zeros
                return 0

            lax.fori_loop(0, S // 16, zstep, 0)
            pltpu.sync_copy(idx_hbm.at[wid], sidx_v)

            def sstep(i, _):
                iv = sidx_v[pl.ds(i * 16, 16)]
                plsc.store_scatter(mask_v, [iv], ones)
                return 0

            lax.fori_loop(0, NK // 16, sstep, 0)
            pltpu.sync_copy(mask_v, out_hbm.at[wid])

    return mask_kernel(idx_bf)


def _proj_body(g_ref, wp_ref, out_ref):
    g = g_ref[0]
    w = wp_ref[0]
    o = lax.dot_general(
        g, w, (((1,), (1,)), ((), ())), preferred_element_type=jnp.float32
    )
    out_ref[...] = o.reshape(out_ref.shape)


def _proj_call(g2, wp, B, k):
    NH, Bk, H = g2.shape
    HD = wp.shape[1]
    return pl.pallas_call(
        _proj_body,
        grid=(NH,),
        in_specs=[
            pl.BlockSpec((1, Bk, H), lambda h: (h, 0, 0)),
            pl.BlockSpec((1, HD, H), lambda h: (h, 0, 0)),
        ],
        out_specs=pl.BlockSpec((B, 1, k, HD), lambda h: (0, h, 0, 0)),
        out_shape=jax.ShapeDtypeStruct((B, NH, k, HD), jnp.float32),
    )(g2, wp)


def kernel(hidden_states, W_proj, W_gate):
    B, S, H = hidden_states.shape
    NH = W_gate.shape[0]
    HD = W_proj.shape[0] // NH
    K = min(64, S)

    gate_bns = _gate_call(hidden_states, W_gate)
    gate_rows = gate_bns.reshape(B * NH, S)

    g2, topk_idx = _topk_gather_mask_call(
        gate_rows, hidden_states.reshape(B * S, H), K, B, NH
    )

    mask_f = _mask_call(topk_idx.reshape(B, NH * K), S)
    index_mask = mask_f != 0.0

    out4 = _proj_call(g2, W_proj.reshape(NH, HD, H), B, K)
    return out4.reshape(B, NH * K, HD), index_mask
